# Initial kernel scaffold; baseline (speedup 1.0000x reference)
import jax
import jax.numpy as jnp
from jax.experimental import pallas as pl


def kernel(ics, weight):
    raise NotImplementedError("write your pallas kernel here")


# speedup vs baseline: 88.7356x; 88.7356x over previous
import functools

import jax
import jax.numpy as jnp
from jax import lax
from jax.experimental import pallas as pl
from jax.experimental.pallas import tpu as pltpu
from jax.experimental.pallas import tpu_sc as plsc

N_FEATURES = 40960
BATCH = 16384
L = 32

NUM_CORES = 2
NUM_SUBCORES = 16
LANES = 16
NUM_WORKERS = NUM_CORES * NUM_SUBCORES
ROWS_PER_W = BATCH // NUM_WORKERS
IDX_PER_W = ROWS_PER_W * L
TBL_PAD = ((N_FEATURES + 1 + LANES - 1) // LANES) * LANES


def _psqt_body(ics_hbm, tbl_hbm, out_hbm, idx_v, tbl_v, out_v, sem_i, sem_t):
    wid = lax.axis_index("s") * NUM_CORES + lax.axis_index("c")
    base = wid * IDX_PER_W

    cp_i = pltpu.async_copy(ics_hbm.at[pl.ds(base, IDX_PER_W)], idx_v, sem_i)
    cp_t = pltpu.async_copy(tbl_hbm, tbl_v, sem_t)
    cp_i.wait()
    cp_t.wait()

    lanes = lax.iota(jnp.int32, 16)

    def group(g, carry):
        ptr0 = (g * 16 + lanes) * L
        acc = jnp.zeros((16,), jnp.float32)
        for l in range(L):
            idx = plsc.load_gather(idx_v, [ptr0 + l])
            acc = acc + plsc.load_gather(tbl_v, [idx + 1])
        out_v[pl.ds(g * 16, 16)] = acc
        return carry

    lax.fori_loop(0, ROWS_PER_W // 16, group, 0)
    pltpu.sync_copy(out_v, out_hbm.at[pl.ds(wid * ROWS_PER_W, ROWS_PER_W)])


@jax.jit
def kernel(ics, weight):
    ics_flat = ics.reshape(BATCH * L)
    tbl = jnp.pad(weight.reshape(N_FEATURES + 1), (0, TBL_PAD - (N_FEATURES + 1)))
    mesh = plsc.VectorSubcoreMesh(core_axis_name="c", subcore_axis_name="s")
    out = pl.kernel(
        _psqt_body,
        out_type=jax.ShapeDtypeStruct((BATCH,), jnp.float32),
        mesh=mesh,
        scratch_types=[
            pltpu.VMEM((IDX_PER_W,), jnp.int32),
            pltpu.VMEM((TBL_PAD,), jnp.float32),
            pltpu.VMEM((ROWS_PER_W,), jnp.float32),
            pltpu.SemaphoreType.DMA,
            pltpu.SemaphoreType.DMA,
        ],
        compiler_params=pltpu.CompilerParams(needs_layout_passes=False),
    )(ics_flat, tbl)
    return out.reshape(BATCH, 1)


# speedup vs baseline: 98.3484x; 1.1083x over previous
import functools

import jax
import jax.numpy as jnp
from jax import lax
from jax.experimental import pallas as pl
from jax.experimental.pallas import tpu as pltpu
from jax.experimental.pallas import tpu_sc as plsc

N_FEATURES = 40960
BATCH = 16384
L = 32

NUM_CORES = 2
NUM_SUBCORES = 16
NUM_WORKERS = NUM_CORES * NUM_SUBCORES
ROWS_PER_W = BATCH // NUM_WORKERS
TBL_PAD = 40976
PITCH = 17


def _psqt_body(ics_hbm, tbl_hbm, out_hbm, idx_v, tbl_v, out_v, skew_v,
               sem_i, sem_t):
    wid = lax.axis_index("s") * NUM_CORES + lax.axis_index("c")
    row_base = wid * ROWS_PER_W

    cp_i = pltpu.async_copy(ics_hbm.at[pl.ds(row_base, ROWS_PER_W), :], idx_v,
                            sem_i)
    cp_t = pltpu.async_copy(tbl_hbm, tbl_v, sem_t)
    cp_i.wait()
    cp_t.wait()

    lanes = lax.iota(jnp.int32, 16)

    def group(g, carry):
        r0 = g * 16
        for r in range(16):
            i0 = idx_v[r0 + r, pl.ds(0, 16)]
            i1 = idx_v[r0 + r, pl.ds(16, 16)]
            w = plsc.load_gather(tbl_v, [i0 + 1]) + plsc.load_gather(
                tbl_v, [i1 + 1])
            plsc.store_scatter(skew_v, [lanes * PITCH + r], w)
        acc = plsc.load_gather(skew_v, [lanes])
        for t in range(1, 16):
            acc = acc + plsc.load_gather(skew_v, [lanes + t * PITCH])
        out_v[pl.ds(r0, 16)] = acc
        return carry

    lax.fori_loop(0, ROWS_PER_W // 16, group, 0)
    pltpu.sync_copy(out_v, out_hbm.at[pl.ds(row_base, ROWS_PER_W)])


@jax.jit
def kernel(ics, weight):
    tbl = jnp.pad(weight.reshape(N_FEATURES + 1), (0, TBL_PAD - (N_FEATURES + 1)))
    mesh = plsc.VectorSubcoreMesh(core_axis_name="c", subcore_axis_name="s")
    out = pl.kernel(
        _psqt_body,
        out_type=jax.ShapeDtypeStruct((BATCH,), jnp.float32),
        mesh=mesh,
        scratch_types=[
            pltpu.VMEM((ROWS_PER_W, L), jnp.int32),
            pltpu.VMEM((TBL_PAD,), jnp.float32),
            pltpu.VMEM((ROWS_PER_W,), jnp.float32),
            pltpu.VMEM((16 * PITCH,), jnp.float32),
            pltpu.SemaphoreType.DMA,
            pltpu.SemaphoreType.DMA,
        ],
        compiler_params=pltpu.CompilerParams(needs_layout_passes=False),
    )(ics, tbl)
    return out.reshape(BATCH, 1)


# speedup vs baseline: 106.4632x; 1.0825x over previous
import functools

import jax
import jax.numpy as jnp
from jax import lax
from jax.experimental import pallas as pl
from jax.experimental.pallas import tpu as pltpu
from jax.experimental.pallas import tpu_sc as plsc

N_FEATURES = 40960
BATCH = 16384
L = 32

NUM_CORES = 2
NUM_SUBCORES = 16
NUM_WORKERS = NUM_CORES * NUM_SUBCORES
ROWS_PER_W = BATCH // NUM_WORKERS
TBL_PAD = 40976
PITCH = ROWS_PER_W + 9
ROW_UNROLL = 8


def _psqt_body(ics_hbm, tbl_hbm, out_hbm, idx_v, tbl_v, out_v, col_v,
               sem_i, sem_t):
    wid = lax.axis_index("s") * NUM_CORES + lax.axis_index("c")
    row_base = wid * ROWS_PER_W

    cp_t = pltpu.async_copy(tbl_hbm, tbl_v, sem_t)
    cp_i = pltpu.async_copy(ics_hbm.at[pl.ds(row_base, ROWS_PER_W), :], idx_v,
                            sem_i)
    cp_i.wait()

    lanes = lax.iota(jnp.int32, 16)
    lo_ptr = lanes * PITCH
    hi_ptr = lo_ptr + 16 * PITCH

    def transpose(it, carry):
        for k in range(ROW_UNROLL):
            r = it * ROW_UNROLL + k
            i0 = idx_v[r, pl.ds(0, 16)] + 1
            i1 = idx_v[r, pl.ds(16, 16)] + 1
            plsc.store_scatter(col_v, [lo_ptr + r], i0)
            plsc.store_scatter(col_v, [hi_ptr + r], i1)
        return carry

    lax.fori_loop(0, ROWS_PER_W // ROW_UNROLL, transpose, 0)
    cp_t.wait()

    def group(g, carry):
        base = g * 16
        acc = [jnp.zeros((16,), jnp.float32) for _ in range(4)]
        for l in range(L):
            idx = col_v[pl.ds(l * PITCH + base, 16)]
            acc[l % 4] = acc[l % 4] + plsc.load_gather(tbl_v, [idx])
        out_v[pl.ds(base, 16)] = (acc[0] + acc[1]) + (acc[2] + acc[3])
        return carry

    lax.fori_loop(0, ROWS_PER_W // 16, group, 0)
    pltpu.sync_copy(out_v, out_hbm.at[pl.ds(row_base, ROWS_PER_W)])


@jax.jit
def kernel(ics, weight):
    tbl = jnp.pad(weight.reshape(N_FEATURES + 1), (0, TBL_PAD - (N_FEATURES + 1)))
    mesh = plsc.VectorSubcoreMesh(core_axis_name="c", subcore_axis_name="s")
    out = pl.kernel(
        _psqt_body,
        out_type=jax.ShapeDtypeStruct((BATCH,), jnp.float32),
        mesh=mesh,
        scratch_types=[
            pltpu.VMEM((ROWS_PER_W, L), jnp.int32),
            pltpu.VMEM((TBL_PAD,), jnp.float32),
            pltpu.VMEM((ROWS_PER_W,), jnp.float32),
            pltpu.VMEM((L * PITCH,), jnp.int32),
            pltpu.SemaphoreType.DMA,
            pltpu.SemaphoreType.DMA,
        ],
        compiler_params=pltpu.CompilerParams(needs_layout_passes=False),
    )(ics, tbl)
    return out.reshape(BATCH, 1)


# speedup vs baseline: 112.3446x; 1.0552x over previous
import functools

import jax
import jax.numpy as jnp
from jax import lax
from jax.experimental import pallas as pl
from jax.experimental.pallas import tpu as pltpu
from jax.experimental.pallas import tpu_sc as plsc

N_FEATURES = 40960
BATCH = 16384
L = 32

NUM_CORES = 2
NUM_SUBCORES = 16
NUM_WORKERS = NUM_CORES * NUM_SUBCORES
ROWS_PER_W = BATCH // NUM_WORKERS
TBL_PAD = 40976


def _psqt_body(ics_hbm, tbl_hbm, out_hbm, idx_v, tbl_v, out_v, sem_i, sem_t):
    wid = lax.axis_index("s") * NUM_CORES + lax.axis_index("c")
    row_base = wid * ROWS_PER_W

    cp_t = pltpu.async_copy(tbl_hbm, tbl_v, sem_t)
    cp_i = pltpu.async_copy(ics_hbm.at[pl.ds(row_base, ROWS_PER_W), :], idx_v,
                            sem_i)
    cp_i.wait()
    cp_t.wait()

    lanes = lax.iota(jnp.int32, 16)

    def group(g, carry):
        base = g * 16
        acc = jnp.zeros((16,), jnp.float32)
        for k in range(16):
            r = base + k
            i0 = idx_v[r, pl.ds(0, 16)] + 1
            i1 = idx_v[r, pl.ds(16, 16)] + 1
            w = plsc.load_gather(tbl_v, [i0]) + plsc.load_gather(tbl_v, [i1])
            acc = jnp.where(lanes == k, jnp.sum(w), acc)
        out_v[pl.ds(base, 16)] = acc
        return carry

    lax.fori_loop(0, ROWS_PER_W // 16, group, 0)
    pltpu.sync_copy(out_v, out_hbm.at[pl.ds(row_base, ROWS_PER_W)])


@jax.jit
def kernel(ics, weight):
    tbl = jnp.pad(weight.reshape(N_FEATURES + 1), (0, TBL_PAD - (N_FEATURES + 1)))
    mesh = plsc.VectorSubcoreMesh(core_axis_name="c", subcore_axis_name="s")
    out = pl.kernel(
        _psqt_body,
        out_type=jax.ShapeDtypeStruct((BATCH,), jnp.float32),
        mesh=mesh,
        scratch_types=[
            pltpu.VMEM((ROWS_PER_W, L), jnp.int32),
            pltpu.VMEM((TBL_PAD,), jnp.float32),
            pltpu.VMEM((ROWS_PER_W,), jnp.float32),
            pltpu.SemaphoreType.DMA,
            pltpu.SemaphoreType.DMA,
        ],
        compiler_params=pltpu.CompilerParams(needs_layout_passes=False),
    )(ics, tbl)
    return out.reshape(BATCH, 1)
